# baseline (speedup 1.0000x reference)
import functools

import jax
import jax.numpy as jnp
from jax import lax
from jax.experimental import pallas as pl
from jax.experimental.pallas import tpu as pltpu
from jax.experimental.pallas import tpu_sc as plsc

N = 10000
E = 320000
D = 128

T = 16
NP = 10240
NPT = NP // T
NV = NPT // 16
EPT = E // T
GV = EPT // 16
WIN = 20096
NZ = NP // 16


def _matvec_body(x_ref, w_ref, o_ref):
    o_ref[...] = jnp.sum(x_ref[...] * w_ref[...], axis=1)


def _matvec(x, W):
    return pl.pallas_call(
        _matvec_body,
        grid=(10,),
        in_specs=[
            pl.BlockSpec((1024, D), lambda i: (i, 0)),
            pl.BlockSpec((1, D), lambda i: (0, 0)),
        ],
        out_specs=pl.BlockSpec((1024,), lambda i: (i,)),
        out_shape=jax.ShapeDtypeStruct((NP,), jnp.float32),
    )(x, W)


def _sc_body(edge_hbm, y0_hbm, b_hbm,
             out_hbm, part_hbm, g_hbm,
             ed_v, g_v, acc_p, pbuf_v, yc_v, gc_v, dis_v, disq_v,
             cc_v, b_v, sem):
    t = lax.axis_index("s")
    base_n = pl.multiple_of(t * NPT, NPT)
    _ZERO16 = jnp.zeros((16,), jnp.float32)
    _ONE16 = jnp.full((16,), 1.0, jnp.float32)

    def zero_acc():
        @plsc.parallel_loop(0, NZ, 1, unroll=8)
        def _(i):
            acc_p[pl.ds(pl.multiple_of(i * 16, 16), 16)] = _ZERO16

    def publish_and_combine(rezero):
        pltpu.sync_copy(acc_p, part_hbm.at[t])
        plsc.subcore_barrier()
        cps = [pltpu.async_copy(part_hbm.at[k, pl.ds(base_n, NPT)],
                                pbuf_v.at[k], sem) for k in range(T)]
        if rezero:
            zero_acc()
        for cp in cps:
            cp.wait()

    def combined(i):
        sl = pl.ds(i * 16, 16)
        s = pbuf_v[0, sl]
        for k in range(1, T):
            s = s + pbuf_v[k, sl]
        return s

    base_e = t * EPT
    astart = jnp.minimum((base_e // 128) * 128, E - WIN)
    astart = pl.multiple_of(astart, 128)
    off = base_e - astart
    edc = pltpu.async_copy(edge_hbm.at[:, pl.ds(astart, WIN)], ed_v, sem)
    pltpu.sync_copy(y0_hbm.at[pl.ds(base_n, NPT)], yc_v)
    pltpu.sync_copy(b_hbm, b_v)
    zero_acc()
    edc.wait()

    B = 10

    @plsc.parallel_loop(0, GV // B, 1, unroll=2)
    def _(i):
        o = pl.multiple_of(off + i * (16 * B), 16)
        dsts = [ed_v[1, pl.ds(o + k * 16, 16)] for k in range(B)]
        for k in range(B):
            plsc.addupdate_scatter(acc_p, [dsts[k]], _ONE16)
    publish_and_combine(rezero=True)

    for i in range(NV):
        sl = pl.ds(i * 16, 16)
        deg = combined(i) + 1.0
        ii = lax.bitcast_convert_type(deg, jnp.int32)
        ii = 0x5F3759DF - (ii >> 1)
        y = lax.bitcast_convert_type(ii, jnp.float32)
        y = y * (1.5 - 0.5 * deg * y * y)
        y = y * (1.5 - 0.5 * deg * y * y)
        y = y * (1.5 - 0.5 * deg * y * y)
        dis_v[sl] = y
        disq_v[sl] = y * y
        gc_v[sl] = y * yc_v[sl]
    pltpu.sync_copy(gc_v, g_hbm.at[pl.ds(base_n, NPT)])
    plsc.subcore_barrier()
    pltpu.sync_copy(g_hbm, g_v)

    def do_hop(rezero):
        @plsc.parallel_loop(0, GV // B, 1, unroll=2)
        def _(i):
            o = pl.multiple_of(off + i * (16 * B), 16)
            srcs = [ed_v[0, pl.ds(o + k * 16, 16)] for k in range(B)]
            dsts = [ed_v[1, pl.ds(o + k * 16, 16)] for k in range(B)]
            vals = [plsc.load_gather(g_v, [ix]) for ix in srcs]
            for k in range(B):
                plsc.addupdate_scatter(acc_p, [dsts[k]], vals[k])
        publish_and_combine(rezero)

    do_hop(rezero=True)
    for i in range(NV):
        sl = pl.ds(i * 16, 16)
        gc_v[sl] = disq_v[sl] * (combined(i) + gc_v[sl])
    pltpu.sync_copy(gc_v, g_hbm.at[pl.ds(base_n, NPT)])
    plsc.subcore_barrier()
    pltpu.sync_copy(g_hbm, g_v)

    do_hop(rezero=False)
    bvec = b_v[pl.ds(0, 16)]
    for i in range(NV):
        sl = pl.ds(i * 16, 16)
        h2 = dis_v[sl] * (combined(i) + gc_v[sl])
        o = h2 + bvec
        cc_v[sl] = o * o
    pltpu.sync_copy(cc_v, out_hbm.at[pl.ds(base_n, NPT)])


_sc_call = functools.partial(
    pl.kernel,
    out_type=(
        jax.ShapeDtypeStruct((NP,), jnp.float32),
        jax.ShapeDtypeStruct((T, NP), jnp.float32),
        jax.ShapeDtypeStruct((NP,), jnp.float32),
    ),
    mesh=plsc.VectorSubcoreMesh(core_axis_name="c", subcore_axis_name="s",
                                num_cores=1),
    compiler_params=pltpu.CompilerParams(needs_layout_passes=False),
    scratch_types=[
        pltpu.VMEM((2, WIN), jnp.int32),
        pltpu.VMEM((NP,), jnp.float32),
        pltpu.VMEM((NP,), jnp.float32),
        pltpu.VMEM((T, NPT), jnp.float32),
        pltpu.VMEM((NPT,), jnp.float32),
        pltpu.VMEM((NPT,), jnp.float32),
        pltpu.VMEM((NPT,), jnp.float32),
        pltpu.VMEM((NPT,), jnp.float32),
        pltpu.VMEM((NPT,), jnp.float32),
        pltpu.VMEM((16,), jnp.float32),
        pltpu.SemaphoreType.DMA,
    ],
)(_sc_body)


@jax.jit
def kernel(x, edge_index, W, b):
    y0p = _matvec(x, W.reshape(1, D))
    edges = edge_index.astype(jnp.int32)
    b16 = jnp.broadcast_to(b, (16,)).astype(jnp.float32)
    out, _, _ = _sc_call(edges, y0p, b16)
    return out[:N].reshape(N, 1)


# speedup vs baseline: 1.0007x; 1.0007x over previous
import functools

import jax
import jax.numpy as jnp
from jax import lax
from jax.experimental import pallas as pl
from jax.experimental.pallas import tpu as pltpu
from jax.experimental.pallas import tpu_sc as plsc

N = 10000
E = 320000
D = 128

T = 16
NP = 10240
NPT = NP // T
NV = NPT // 16
EPT = E // T
GV = EPT // 16
WIN = 20096
NZ = NP // 16


def _matvec_body(x_ref, w_ref, o_ref):
    o_ref[...] = lax.dot_general(x_ref[...], w_ref[0],
                                 (((1,), (0,)), ((), ())),
                                 preferred_element_type=jnp.float32)


def _matvec(x, W):
    return pl.pallas_call(
        _matvec_body,
        grid=(10,),
        in_specs=[
            pl.BlockSpec((1024, D), lambda i: (i, 0)),
            pl.BlockSpec((1, D), lambda i: (0, 0)),
        ],
        out_specs=pl.BlockSpec((1024,), lambda i: (i,)),
        out_shape=jax.ShapeDtypeStruct((NP,), jnp.float32),
    )(x, W)


def _sc_body(edge_hbm, y0_hbm, b_hbm,
             out_hbm, part_hbm, g_hbm,
             ed_v, g_v, acc_p, pbuf_v, yc_v, gc_v, dis_v, disq_v,
             cc_v, b_v, sem):
    t = lax.axis_index("s")
    base_n = pl.multiple_of(t * NPT, NPT)
    _ZERO16 = jnp.zeros((16,), jnp.float32)
    _ONE16 = jnp.full((16,), 1.0, jnp.float32)

    def zero_acc():
        @plsc.parallel_loop(0, NZ, 1, unroll=8)
        def _(i):
            acc_p[pl.ds(pl.multiple_of(i * 16, 16), 16)] = _ZERO16

    def publish_and_combine(rezero):
        pltpu.sync_copy(acc_p, part_hbm.at[t])
        plsc.subcore_barrier()
        cps = [pltpu.async_copy(part_hbm.at[k, pl.ds(base_n, NPT)],
                                pbuf_v.at[k], sem) for k in range(T)]
        if rezero:
            zero_acc()
        for cp in cps:
            cp.wait()

    def combined(i):
        sl = pl.ds(i * 16, 16)
        s = pbuf_v[0, sl]
        for k in range(1, T):
            s = s + pbuf_v[k, sl]
        return s

    base_e = t * EPT
    astart = jnp.minimum((base_e // 128) * 128, E - WIN)
    astart = pl.multiple_of(astart, 128)
    off = base_e - astart
    edc = pltpu.async_copy(edge_hbm.at[:, pl.ds(astart, WIN)], ed_v, sem)
    pltpu.sync_copy(y0_hbm.at[pl.ds(base_n, NPT)], yc_v)
    pltpu.sync_copy(b_hbm, b_v)
    zero_acc()
    edc.wait()

    B = 10

    @plsc.parallel_loop(0, GV // B, 1, unroll=2)
    def _(i):
        o = pl.multiple_of(off + i * (16 * B), 16)
        dsts = [ed_v[1, pl.ds(o + k * 16, 16)] for k in range(B)]
        for k in range(B):
            plsc.addupdate_scatter(acc_p, [dsts[k]], _ONE16)
    publish_and_combine(rezero=True)

    for i in range(NV):
        sl = pl.ds(i * 16, 16)
        deg = combined(i) + 1.0
        ii = lax.bitcast_convert_type(deg, jnp.int32)
        ii = 0x5F3759DF - (ii >> 1)
        y = lax.bitcast_convert_type(ii, jnp.float32)
        y = y * (1.5 - 0.5 * deg * y * y)
        y = y * (1.5 - 0.5 * deg * y * y)
        y = y * (1.5 - 0.5 * deg * y * y)
        dis_v[sl] = y
        disq_v[sl] = y * y
        gc_v[sl] = y * yc_v[sl]
    pltpu.sync_copy(gc_v, g_hbm.at[pl.ds(base_n, NPT)])
    plsc.subcore_barrier()
    pltpu.sync_copy(g_hbm, g_v)

    def do_hop(rezero):
        @plsc.parallel_loop(0, GV // B, 1, unroll=2)
        def _(i):
            o = pl.multiple_of(off + i * (16 * B), 16)
            srcs = [ed_v[0, pl.ds(o + k * 16, 16)] for k in range(B)]
            dsts = [ed_v[1, pl.ds(o + k * 16, 16)] for k in range(B)]
            vals = [plsc.load_gather(g_v, [ix]) for ix in srcs]
            for k in range(B):
                plsc.addupdate_scatter(acc_p, [dsts[k]], vals[k])
        publish_and_combine(rezero)

    do_hop(rezero=True)
    for i in range(NV):
        sl = pl.ds(i * 16, 16)
        gc_v[sl] = disq_v[sl] * (combined(i) + gc_v[sl])
    pltpu.sync_copy(gc_v, g_hbm.at[pl.ds(base_n, NPT)])
    plsc.subcore_barrier()
    pltpu.sync_copy(g_hbm, g_v)

    do_hop(rezero=False)
    bvec = b_v[pl.ds(0, 16)]
    for i in range(NV):
        sl = pl.ds(i * 16, 16)
        h2 = dis_v[sl] * (combined(i) + gc_v[sl])
        o = h2 + bvec
        cc_v[sl] = o * o
    pltpu.sync_copy(cc_v, out_hbm.at[pl.ds(base_n, NPT)])


_sc_call = functools.partial(
    pl.kernel,
    out_type=(
        jax.ShapeDtypeStruct((NP,), jnp.float32),
        jax.ShapeDtypeStruct((T, NP), jnp.float32),
        jax.ShapeDtypeStruct((NP,), jnp.float32),
    ),
    mesh=plsc.VectorSubcoreMesh(core_axis_name="c", subcore_axis_name="s",
                                num_cores=1),
    compiler_params=pltpu.CompilerParams(needs_layout_passes=False),
    scratch_types=[
        pltpu.VMEM((2, WIN), jnp.int32),
        pltpu.VMEM((NP,), jnp.float32),
        pltpu.VMEM((NP,), jnp.float32),
        pltpu.VMEM((T, NPT), jnp.float32),
        pltpu.VMEM((NPT,), jnp.float32),
        pltpu.VMEM((NPT,), jnp.float32),
        pltpu.VMEM((NPT,), jnp.float32),
        pltpu.VMEM((NPT,), jnp.float32),
        pltpu.VMEM((NPT,), jnp.float32),
        pltpu.VMEM((16,), jnp.float32),
        pltpu.SemaphoreType.DMA,
    ],
)(_sc_body)


@jax.jit
def kernel(x, edge_index, W, b):
    y0p = _matvec(x, W.reshape(1, D))
    edges = edge_index.astype(jnp.int32)
    b16 = jnp.broadcast_to(b, (16,)).astype(jnp.float32)
    out, _, _ = _sc_call(edges, y0p, b16)
    return out[:N].reshape(N, 1)


# speedup vs baseline: 1.0356x; 1.0349x over previous
import functools

import jax
import jax.numpy as jnp
from jax import lax
from jax.experimental import pallas as pl
from jax.experimental.pallas import tpu as pltpu
from jax.experimental.pallas import tpu_sc as plsc

N = 10000
E = 320000
D = 128

T = 16
NP = 10240
NPT = NP // T
NV = NPT // 16
EPT = E // T
GV = EPT // 16
WIN = 20096
NZ = NP // 16


def _matvec_body(x_ref, w_ref, o_ref):
    o_ref[...] = lax.dot_general(x_ref[...], w_ref[0],
                                 (((1,), (0,)), ((), ())),
                                 preferred_element_type=jnp.float32)


def _matvec(x, W):
    return pl.pallas_call(
        _matvec_body,
        grid=(5,),
        in_specs=[
            pl.BlockSpec((2048, D), lambda i: (i, 0)),
            pl.BlockSpec((1, D), lambda i: (0, 0)),
        ],
        out_specs=pl.BlockSpec((2048,), lambda i: (i,)),
        out_shape=jax.ShapeDtypeStruct((NP,), jnp.float32),
    )(x, W)


def _sc_body(edge_hbm, y0_hbm, b_hbm,
             out_hbm, part_hbm, g_hbm,
             ed_v, g_v, acc_p, pbuf_v, yc_v, gc_v, dis_v, disq_v,
             cc_v, b_v, sem):
    t = lax.axis_index("s")
    base_n = pl.multiple_of(t * NPT, NPT)
    _ZERO16 = jnp.zeros((16,), jnp.float32)
    _ONE16 = jnp.full((16,), 1.0, jnp.float32)

    def zero_acc():
        @plsc.parallel_loop(0, NZ, 1, unroll=8)
        def _(i):
            acc_p[pl.ds(pl.multiple_of(i * 16, 16), 16)] = _ZERO16

    def publish_and_combine(rezero):
        pltpu.sync_copy(acc_p, part_hbm.at[t])
        plsc.subcore_barrier()
        cps = [pltpu.async_copy(part_hbm.at[k, pl.ds(base_n, NPT)],
                                pbuf_v.at[k], sem) for k in range(T)]
        if rezero:
            zero_acc()
        for cp in cps:
            cp.wait()

    def combined(i):
        sl = pl.ds(i * 16, 16)
        s = pbuf_v[0, sl]
        for k in range(1, T):
            s = s + pbuf_v[k, sl]
        return s

    base_e = t * EPT
    astart = jnp.minimum((base_e // 128) * 128, E - WIN)
    astart = pl.multiple_of(astart, 128)
    off = base_e - astart
    edc = pltpu.async_copy(edge_hbm.at[:, pl.ds(astart, WIN)], ed_v, sem)
    pltpu.sync_copy(y0_hbm.at[pl.ds(base_n, NPT)], yc_v)
    pltpu.sync_copy(b_hbm, b_v)
    zero_acc()
    edc.wait()

    B = 10

    @plsc.parallel_loop(0, GV // B, 1, unroll=2)
    def _(i):
        o = pl.multiple_of(off + i * (16 * B), 16)
        dsts = [ed_v[1, pl.ds(o + k * 16, 16)] for k in range(B)]
        for k in range(B):
            plsc.addupdate_scatter(acc_p, [dsts[k]], _ONE16)
    publish_and_combine(rezero=True)

    for i in range(NV):
        sl = pl.ds(i * 16, 16)
        deg = combined(i) + 1.0
        ii = lax.bitcast_convert_type(deg, jnp.int32)
        ii = 0x5F3759DF - (ii >> 1)
        y = lax.bitcast_convert_type(ii, jnp.float32)
        y = y * (1.5 - 0.5 * deg * y * y)
        y = y * (1.5 - 0.5 * deg * y * y)
        y = y * (1.5 - 0.5 * deg * y * y)
        dis_v[sl] = y
        disq_v[sl] = y * y
        gc_v[sl] = y * yc_v[sl]
    pltpu.sync_copy(gc_v, g_hbm.at[pl.ds(base_n, NPT)])
    plsc.subcore_barrier()
    pltpu.sync_copy(g_hbm, g_v)

    def do_hop(rezero):
        @plsc.parallel_loop(0, GV // B, 1, unroll=2)
        def _(i):
            o = pl.multiple_of(off + i * (16 * B), 16)
            srcs = [ed_v[0, pl.ds(o + k * 16, 16)] for k in range(B)]
            dsts = [ed_v[1, pl.ds(o + k * 16, 16)] for k in range(B)]
            vals = [plsc.load_gather(g_v, [ix]) for ix in srcs]
            for k in range(B):
                plsc.addupdate_scatter(acc_p, [dsts[k]], vals[k])
        publish_and_combine(rezero)

    do_hop(rezero=True)
    for i in range(NV):
        sl = pl.ds(i * 16, 16)
        gc_v[sl] = disq_v[sl] * (combined(i) + gc_v[sl])
    pltpu.sync_copy(gc_v, g_hbm.at[pl.ds(base_n, NPT)])
    plsc.subcore_barrier()
    pltpu.sync_copy(g_hbm, g_v)

    do_hop(rezero=False)
    bvec = b_v[pl.ds(0, 16)]
    for i in range(NV):
        sl = pl.ds(i * 16, 16)
        h2 = dis_v[sl] * (combined(i) + gc_v[sl])
        o = h2 + bvec
        cc_v[sl] = o * o
    pltpu.sync_copy(cc_v, out_hbm.at[pl.ds(base_n, NPT)])


_sc_call = functools.partial(
    pl.kernel,
    out_type=(
        jax.ShapeDtypeStruct((NP,), jnp.float32),
        jax.ShapeDtypeStruct((T, NP), jnp.float32),
        jax.ShapeDtypeStruct((NP,), jnp.float32),
    ),
    mesh=plsc.VectorSubcoreMesh(core_axis_name="c", subcore_axis_name="s",
                                num_cores=1),
    compiler_params=pltpu.CompilerParams(needs_layout_passes=False),
    scratch_types=[
        pltpu.VMEM((2, WIN), jnp.int32),
        pltpu.VMEM((NP,), jnp.float32),
        pltpu.VMEM((NP,), jnp.float32),
        pltpu.VMEM((T, NPT), jnp.float32),
        pltpu.VMEM((NPT,), jnp.float32),
        pltpu.VMEM((NPT,), jnp.float32),
        pltpu.VMEM((NPT,), jnp.float32),
        pltpu.VMEM((NPT,), jnp.float32),
        pltpu.VMEM((NPT,), jnp.float32),
        pltpu.VMEM((16,), jnp.float32),
        pltpu.SemaphoreType.DMA,
    ],
)(_sc_body)


@jax.jit
def kernel(x, edge_index, W, b):
    y0p = _matvec(x, W.reshape(1, D))
    edges = edge_index.astype(jnp.int32)
    b16 = jnp.broadcast_to(b, (16,)).astype(jnp.float32)
    out, _, _ = _sc_call(edges, y0p, b16)
    return out[:N].reshape(N, 1)
